# Initial kernel scaffold; baseline (speedup 1.0000x reference)
import jax
import jax.numpy as jnp
from jax.experimental import pallas as pl


def kernel(v1, v2, cmap, rid_to_vid_list):
    raise NotImplementedError("write your pallas kernel here")


# speedup vs baseline: 65.1129x; 65.1129x over previous
import functools

import jax
import jax.numpy as jnp
from jax import lax
from jax.experimental import pallas as pl
from jax.experimental.pallas import tpu as pltpu
from jax.experimental.pallas import tpu_sc as plsc

B = 8
NV = 6890
R = 75
MV = 40
NR = R * MV
NRP = 3008
KF = 16

NW = 32
PER_W = 1536
TOT = NW * PER_W
HALF = TOT // 2
CH = 128


def _sc_gather(table, idx):
    mesh = plsc.VectorSubcoreMesh(core_axis_name="c", subcore_axis_name="s")

    @functools.partial(
        pl.kernel,
        out_type=jax.ShapeDtypeStruct((TOT, KF), jnp.float32),
        mesh=mesh,
        scratch_types=[
            pltpu.VMEM((PER_W,), jnp.int32),
            pltpu.VMEM((PER_W, KF), jnp.float32),
            pltpu.SemaphoreType.DMA,
        ],
        compiler_params=pltpu.CompilerParams(use_tc_tiling_on_sc=False),
    )
    def gather_kernel(table_hbm, idx_hbm, out_hbm, idx_v, rows_v, sem):
        wid = lax.axis_index("s") * 2 + lax.axis_index("c")
        base = wid * PER_W
        pltpu.sync_copy(idx_hbm.at[pl.ds(base, PER_W)], idx_v)
        for j in range(0, PER_W, CH):
            pltpu.async_copy(
                table_hbm.at[idx_v.at[pl.ds(j, CH)]],
                rows_v.at[pl.ds(j, CH)],
                sem,
            )
        for j in range(0, PER_W, CH):
            pltpu.make_async_copy(
                table_hbm.at[idx_v.at[pl.ds(j, CH)]],
                rows_v.at[pl.ds(j, CH)],
                sem,
            ).wait()
        pltpu.sync_copy(rows_v, out_hbm.at[pl.ds(base, PER_W)])

    return gather_kernel(table, idx)


def _dense_body(g1, g2, g1t, g2t, w1, w2, out, b1aug, b2aug):
    h = pl.program_id(1)

    @pl.when(h == 0)
    def _build_baug():
        for src, dst in ((g1t, b1aug), (g2t, b2aug)):
            coords = src[0][:3, :]
            yy = jnp.sum(coords * coords, axis=0, keepdims=True)
            ones = jnp.ones_like(yy)
            zeros = jnp.zeros((KF - 5, NRP), jnp.float32)
            dst[...] = jnp.concatenate([coords, ones, yy, zeros], axis=0)
        out[...] = jnp.zeros_like(out)

    def half(a_ref, baug, w):
        a = a_ref[0]
        ac = a[:, :3]
        xx = jnp.sum(ac * ac, axis=1, keepdims=True)
        ones = jnp.ones_like(xx)
        zeros = jnp.zeros((MV, KF - 5), jnp.float32)
        aaug = jnp.concatenate([-2.0 * ac, xx, ones, zeros], axis=1)
        d2 = jnp.dot(aaug, baug[...], preferred_element_type=jnp.float32)
        d2 = jnp.maximum(d2, 1e-12)
        cmin = jnp.min(d2, axis=0, keepdims=True)
        mask = (w[0] != 0.0).astype(jnp.float32)
        return jnp.sum(cmin * mask)

    contrib = half(g1, b2aug, w1) + half(g2, b1aug, w2)
    out[...] += contrib * (1.0 / MV)


def _dense_call(g1r, g2r, g1t, g2t, w1, w2):
    return pl.pallas_call(
        _dense_body,
        grid=(B, R),
        in_specs=[
            pl.BlockSpec((1, MV, KF), lambda b, h: (b, h, 0)),
            pl.BlockSpec((1, MV, KF), lambda b, h: (b, h, 0)),
            pl.BlockSpec((1, KF, NRP), lambda b, h: (b, 0, 0)),
            pl.BlockSpec((1, KF, NRP), lambda b, h: (b, 0, 0)),
            pl.BlockSpec((1, 1, NRP), lambda b, h: (b * R + h, 0, 0)),
            pl.BlockSpec((1, 1, NRP), lambda b, h: (b * R + h, 0, 0)),
        ],
        out_specs=pl.BlockSpec((1, 1, 128), lambda b, h: (b, 0, 0)),
        out_shape=jax.ShapeDtypeStruct((B, 1, 128), jnp.float32),
        scratch_shapes=[
            pltpu.VMEM((KF, NRP), jnp.float32),
            pltpu.VMEM((KF, NRP), jnp.float32),
        ],
        compiler_params=pltpu.CompilerParams(
            dimension_semantics=("arbitrary", "arbitrary")),
    )(g1r, g2r, g1t, g2t, w1, w2)


def kernel(v1, v2, cmap, rid_to_vid_list):
    f32 = jnp.float32
    v1 = v1.astype(f32)
    v2 = v2.astype(f32)

    t = jnp.concatenate([v1.reshape(B * NV, 3), v2.reshape(B * NV, 3)], axis=0)
    t = jnp.pad(t, ((0, 0), (0, KF - 3)))

    rid = rid_to_vid_list.reshape(-1).astype(jnp.int32)
    boff = (jnp.arange(B, dtype=jnp.int32) * NV)[:, None]
    idx1 = (boff + rid[None, :]).reshape(-1)
    idx1 = jnp.pad(idx1, (0, HALF - NR * B))
    idx2 = idx1 + (B * NV)
    idx = jnp.concatenate([idx1, idx2], axis=0)

    g = _sc_gather(t, idx)
    g1 = g[: B * NR].reshape(B, NR, KF)
    g2 = g[HALF : HALF + B * NR].reshape(B, NR, KF)
    g1t = jnp.pad(g1.transpose(0, 2, 1), ((0, 0), (0, 0), (0, NRP - NR)))
    g2t = jnp.pad(g2.transpose(0, 2, 1), ((0, 0), (0, 0), (0, NRP - NR)))

    w1 = jnp.pad(jnp.repeat(cmap, MV, axis=2), ((0, 0), (0, 0), (0, NRP - NR)))
    w2 = jnp.pad(jnp.repeat(cmap.transpose(0, 2, 1), MV, axis=2),
                 ((0, 0), (0, 0), (0, NRP - NR)))
    w1 = w1.reshape(B * R, 1, NRP)
    w2 = w2.reshape(B * R, 1, NRP)

    out = _dense_call(g1, g2, g1t, g2t, w1, w2)
    return out[:, 0, 0]
